# baseline (speedup 1.0000x reference)
import functools

import jax
import jax.numpy as jnp
from jax import lax
from jax.experimental import pallas as pl
from jax.experimental.pallas import tpu as pltpu
from jax.experimental.pallas import tpu_sc as plsc

_B, _L, _C = 32, 2101, 64
_S = 64
_NB = 32
_W = _NB * _S
_PAD = _L - _W
_NC, _NS = 2, 16
_CHUNK = 256
_NCHUNK = _W // _CHUNK
_OCHUNK = 256
_Q = _C // 16


def _sq_row_acc(obuf, ybuf, base, r, acc):
    new = []
    for q in range(_Q):
        o = obuf[base + r, pl.ds(q * 16, 16)]
        y = ybuf[base + r, pl.ds(q * 16, 16)]
        d = o - y
        new.append(acc[q] + d * d)
    return tuple(new)


def _sc_body(o_hbm, y_hbm, out_hbm, obuf, ybuf, sbuf, vstage, vbuf):
    cid = lax.axis_index("c")
    sid = lax.axis_index("s")
    b = sid * _NC + cid
    zero4 = (jnp.zeros((16,), jnp.float32),) * _Q

    for k in range(_NCHUNK):
        pltpu.sync_copy(o_hbm.at[b, pl.ds(k * _CHUNK, _CHUNK), :], obuf)
        pltpu.sync_copy(y_hbm.at[b, pl.ds(k * _CHUNK, _CHUNK), :], ybuf)
        for jj in range(_CHUNK // _S):
            j = k * (_CHUNK // _S) + jj
            body = functools.partial(_sq_row_acc, obuf, ybuf, jj * _S)
            acc = lax.fori_loop(0, _S, body, zero4)
            for q in range(_Q):
                sbuf[j, pl.ds(q * 16, 16)] = acc[q]
    pltpu.sync_copy(o_hbm.at[b, pl.ds(_W, _PAD), :], obuf.at[pl.ds(0, _PAD)])
    pltpu.sync_copy(y_hbm.at[b, pl.ds(_W, _PAD), :], ybuf.at[pl.ds(0, _PAD)])
    acc = lax.fori_loop(0, _PAD, functools.partial(_sq_row_acc, obuf, ybuf, 0),
                        zero4)
    for q in range(_Q):
        sbuf[_NB, pl.ds(q * 16, 16)] = acc[q]

    for q in range(_Q):
        lanes = pl.ds(q * 16, 16)
        s_prev = sbuf[0, lanes]
        s_cur = sbuf[1, lanes]
        mp_prev = s_prev + s_cur
        nz_prev = jnp.sign(mp_prev)
        vstage[0, lanes] = mp_prev / nz_prev
        for j in range(1, _NB - 1):
            s_next = sbuf[j + 1, lanes]
            mp_cur = s_cur + s_next
            nz_cur = jnp.sign(mp_cur)
            vstage[j, lanes] = (mp_prev + mp_cur) / (nz_prev + nz_cur)
            mp_prev, nz_prev, s_cur = mp_cur, nz_cur, s_next
        vstage[_NB - 1, lanes] = mp_prev / nz_prev

    for g in range(_W // _OCHUNK):
        for jj in range(_OCHUNK // _S):
            j = g * (_OCHUNK // _S) + jj
            vals = [vstage[j, pl.ds(q * 16, 16)] for q in range(_Q)]

            def row_store(r, carry, _jj=jj, _vals=vals):
                for q in range(_Q):
                    vbuf[_jj * _S + r, pl.ds(q * 16, 16)] = _vals[q]
                return carry
            lax.fori_loop(0, _S, row_store, 0)
        pltpu.sync_copy(vbuf, out_hbm.at[b, pl.ds(g * _OCHUNK, _OCHUNK), :])
    tvals = [sbuf[_NB, pl.ds(q * 16, 16)] for q in range(_Q)]

    def tail_store(r, carry):
        for q in range(_Q):
            vbuf[r, pl.ds(q * 16, 16)] = tvals[q]
        return carry
    lax.fori_loop(0, _PAD, tail_store, 0)
    pltpu.sync_copy(vbuf.at[pl.ds(0, _PAD)], out_hbm.at[b, pl.ds(_W, _PAD), :])


def kernel(outputs, batch_y):
    mesh = plsc.VectorSubcoreMesh(core_axis_name="c", subcore_axis_name="s",
                                  num_cores=_NC, num_subcores=_NS)
    run = pl.kernel(
        _sc_body,
        out_type=jax.ShapeDtypeStruct((_B, _L, _C), jnp.float32),
        mesh=mesh,
        scratch_types=[
            pltpu.VMEM((_CHUNK, _C), jnp.float32),
            pltpu.VMEM((_CHUNK, _C), jnp.float32),
            pltpu.VMEM((_NB + 1, _C), jnp.float32),
            pltpu.VMEM((_NB, _C), jnp.float32),
            pltpu.VMEM((_OCHUNK, _C), jnp.float32),
        ],
        compiler_params=pltpu.CompilerParams(use_tc_tiling_on_sc=True),
    )
    return run(outputs, batch_y)


# speedup vs baseline: 1.4443x; 1.4443x over previous
import jax
import jax.numpy as jnp
from jax.experimental import pallas as pl
from jax.experimental.pallas import tpu as pltpu

_B, _L, _C = 32, 2101, 64
_S = 64
_NB = 32
_W = _NB * _S
_PAD = _L - _W
_NBUF = 6


def _compute(o, y):
    d = o - y
    sq = d * d
    main = sq[:_W].reshape(_NB, _S, _C)
    s = jnp.sum(main, axis=1)
    tail = jnp.sum(sq[_W:], axis=0, keepdims=True)
    mp = s[:-1] + s[1:]
    nz = (mp != 0).astype(jnp.float32)
    num = jnp.concatenate([mp[:1], mp[:-1] + mp[1:], mp[-1:]], axis=0)
    cnt = jnp.concatenate([nz[:1], nz[:-1] + nz[1:], nz[-1:]], axis=0)
    v = num / cnt
    body = jnp.broadcast_to(v[:, None, :], (_NB, _S, _C)).reshape(_W, _C)
    tail_b = jnp.broadcast_to(tail, (_PAD, _C))
    return jnp.concatenate([body, tail_b], axis=0)


def _man_kernel(o_hbm, y_hbm, out_hbm, obuf, ybuf, vbuf, isem, osem):
    def start_in(b, slot):
        pltpu.make_async_copy(o_hbm.at[b], obuf.at[slot], isem.at[slot, 0]).start()
        pltpu.make_async_copy(y_hbm.at[b], ybuf.at[slot], isem.at[slot, 1]).start()

    for b in range(_NBUF):
        start_in(b, b)
    for b in range(_B):
        slot = b % _NBUF
        pltpu.make_async_copy(o_hbm.at[b], obuf.at[slot], isem.at[slot, 0]).wait()
        pltpu.make_async_copy(y_hbm.at[b], ybuf.at[slot], isem.at[slot, 1]).wait()
        res = _compute(obuf[slot], ybuf[slot])
        if b >= _NBUF:
            pltpu.make_async_copy(vbuf.at[slot], out_hbm.at[b - _NBUF],
                                  osem.at[slot]).wait()
        vbuf[slot] = res
        pltpu.make_async_copy(vbuf.at[slot], out_hbm.at[b], osem.at[slot]).start()
        if b + _NBUF < _B:
            start_in(b + _NBUF, slot)
    for b in range(_B - _NBUF, _B):
        slot = b % _NBUF
        pltpu.make_async_copy(vbuf.at[slot], out_hbm.at[b], osem.at[slot]).wait()


def kernel(outputs, batch_y):
    return pl.pallas_call(
        _man_kernel,
        in_specs=[
            pl.BlockSpec(memory_space=pl.ANY),
            pl.BlockSpec(memory_space=pl.ANY),
        ],
        out_specs=pl.BlockSpec(memory_space=pl.ANY),
        out_shape=jax.ShapeDtypeStruct((_B, _L, _C), jnp.float32),
        scratch_shapes=[
            pltpu.VMEM((_NBUF, _L, _C), jnp.float32),
            pltpu.VMEM((_NBUF, _L, _C), jnp.float32),
            pltpu.VMEM((_NBUF, _L, _C), jnp.float32),
            pltpu.SemaphoreType.DMA((_NBUF, 2)),
            pltpu.SemaphoreType.DMA((_NBUF,)),
        ],
    )(outputs, batch_y)
